# baseline (speedup 1.0000x reference)
import jax
import jax.numpy as jnp
from jax import lax
from jax.experimental import pallas as pl
from jax.experimental.pallas import tpu as pltpu
from jax.experimental.pallas import tpu_sc as plsc

N_USER = 10000
N_ITEM = 10000
E = 320000
D = 128
CNT_W = 16

NC, NS = 2, 16
NW = NC * NS
E_PER_W = E // NW
CHUNK = 64
NCHUNKP = 158
PAD_E = NCHUNKP * CHUNK - E_PER_W
NHALF = NCHUNKP // 2
DUMMY = N_USER
ACC_ROWS = N_USER + 16
RPT = 624
TAIL = N_USER - NS * RPT
TAIL0 = NS * RPT


def _sc_body(feat_user, feat_item, z_s, z_c, src_f, dst_f, src_c, dst_c,
             src_b, dst_b,
             out_sf, out_cf, out_sc, out_cc, out_sb, out_cb,
             acc_s, acc_c, sidx, didx, rows0, ones, g0, g1, s0, s1):
    c = lax.axis_index("c")
    s = lax.axis_index("s")
    wid = c * NS + s
    row0 = pl.multiple_of(s * RPT, 8)

    lane = lax.iota(jnp.int32, 16)
    one_vec = jnp.where(lane == 0, 1.0, 0.0).astype(jnp.float32)

    def fill_ones(i, carry):
        ones[i, :] = one_vec
        return carry

    lax.fori_loop(0, CHUNK, fill_ones, 0)

    def run_relation(feat, src_r, dst_r, out_s, out_c):
        d1 = pltpu.async_copy(z_s, acc_s.at[pl.ds(row0, RPT)], g0)
        d2 = pltpu.async_copy(z_c, acc_c.at[pl.ds(row0, RPT)], g1)
        d3 = pltpu.async_copy(src_r.at[wid], sidx, s0)
        d4 = pltpu.async_copy(dst_r.at[wid], didx, s1)

        @pl.when(s == 0)
        def _zero_tail():
            pltpu.async_copy(
                z_s.at[pl.ds(0, TAIL)], acc_s.at[pl.ds(TAIL0, TAIL)], g0).wait()
            pltpu.async_copy(
                z_c.at[pl.ds(0, TAIL)], acc_c.at[pl.ds(TAIL0, TAIL)], g1).wait()

        d1.wait()
        d2.wait()
        d3.wait()
        d4.wait()
        plsc.subcore_barrier()

        def g_issue(buf, j, sem):
            pltpu.async_copy(feat.at[sidx.at[j]], buf, sem)

        def g_wait(buf, sem):
            pltpu.make_async_copy(feat.at[sidx.at[0]], buf, sem).wait()

        def s_issue(buf, j, sem):
            pltpu.async_copy(buf, acc_s.at[didx.at[j]], sem, add=True)
            pltpu.async_copy(ones, acc_c.at[didx.at[j]], sem, add=True)

        def s_wait(buf, sem):
            pltpu.make_async_copy(buf, acc_s.at[didx.at[0]], sem).wait()
            pltpu.make_async_copy(ones, acc_c.at[didx.at[0]], sem).wait()

        def chunk(j, carry):
            pltpu.async_copy(feat.at[sidx.at[j]], rows0, g0).wait()
            pltpu.sync_copy(rows0, acc_s.at[didx.at[j]], add=True)
            pltpu.sync_copy(ones, acc_c.at[didx.at[j]], add=True)
            return carry

        lax.fori_loop(0, NCHUNKP, chunk, 0)

        plsc.subcore_barrier()
        pltpu.sync_copy(acc_s.at[pl.ds(row0, RPT)],
                        out_s.at[c, pl.ds(row0, RPT)])
        pltpu.sync_copy(acc_c.at[pl.ds(row0, RPT)],
                        out_c.at[c, pl.ds(row0, RPT)])

        @pl.when(s == 0)
        def _out_tail():
            pltpu.sync_copy(acc_s.at[pl.ds(TAIL0, TAIL)],
                            out_s.at[c, pl.ds(TAIL0, TAIL)])
            pltpu.sync_copy(acc_c.at[pl.ds(TAIL0, TAIL)],
                            out_c.at[c, pl.ds(TAIL0, TAIL)])

        plsc.subcore_barrier()

    run_relation(feat_user, src_f, dst_f, out_sf, out_cf)
    run_relation(feat_user, src_c, dst_c, out_sc, out_cc)
    run_relation(feat_item, src_b, dst_b, out_sb, out_cb)


_sc_agg = pl.kernel(
    _sc_body,
    out_type=[
        jax.ShapeDtypeStruct((NC, N_USER, D), jnp.float32),
        jax.ShapeDtypeStruct((NC, N_USER, CNT_W), jnp.float32),
        jax.ShapeDtypeStruct((NC, N_ITEM, D), jnp.float32),
        jax.ShapeDtypeStruct((NC, N_ITEM, CNT_W), jnp.float32),
        jax.ShapeDtypeStruct((NC, N_USER, D), jnp.float32),
        jax.ShapeDtypeStruct((NC, N_USER, CNT_W), jnp.float32),
    ],
    mesh=plsc.VectorSubcoreMesh(core_axis_name="c", subcore_axis_name="s"),
    compiler_params=pltpu.CompilerParams(use_tc_tiling_on_sc=False),
    scratch_types=[
        pltpu.VMEM_SHARED((ACC_ROWS, D), jnp.float32),
        pltpu.VMEM_SHARED((ACC_ROWS, CNT_W), jnp.float32),
        pltpu.VMEM((NCHUNKP, CHUNK), jnp.int32),
        pltpu.VMEM((NCHUNKP, CHUNK), jnp.int32),
        pltpu.VMEM((CHUNK, D), jnp.float32),
        pltpu.VMEM((CHUNK, CNT_W), jnp.float32),
        pltpu.SemaphoreType.DMA,
        pltpu.SemaphoreType.DMA,
        pltpu.SemaphoreType.DMA,
        pltpu.SemaphoreType.DMA,
    ],
)

BR = 400
NBLK = N_USER // BR


def _tc_body(sf, cf, sb, cb, sc_r, cc, wf, bf, wb, bb, wc, bc, hu, hi):
    def rel(sref, cref, wref, bref):
        tot = sref[0] + sref[1]
        cnt = cref[0, :, 0:1] + cref[1, :, 0:1]
        mean = tot / jnp.maximum(cnt, 1.0)
        h = lax.dot_general(mean, wref[...], (((1,), (1,)), ((), ())),
                            preferred_element_type=jnp.float32,
                            precision=lax.Precision.HIGHEST) + bref[...]
        return jnp.where(cnt > 0, h, 0.0)

    hu[...] = rel(sf, cf, wf, bf) + rel(sb, cb, wb, bb)
    hi[...] = rel(sc_r, cc, wc, bc)


_sum_spec = pl.BlockSpec((NC, BR, D), lambda i: (0, i, 0))
_cnt_spec = pl.BlockSpec((NC, BR, CNT_W), lambda i: (0, i, 0))
_w_spec = pl.BlockSpec((D, D), lambda i: (0, 0))
_b_spec = pl.BlockSpec((1, D), lambda i: (0, 0))

_tc_finish = pl.pallas_call(
    _tc_body,
    grid=(NBLK,),
    in_specs=[_sum_spec, _cnt_spec, _sum_spec, _cnt_spec, _sum_spec, _cnt_spec,
              _w_spec, _b_spec, _w_spec, _b_spec, _w_spec, _b_spec],
    out_specs=[pl.BlockSpec((BR, D), lambda i: (i, 0)),
               pl.BlockSpec((BR, D), lambda i: (i, 0))],
    out_shape=[jax.ShapeDtypeStruct((N_USER, D), jnp.float32),
               jax.ShapeDtypeStruct((N_ITEM, D), jnp.float32)],
)


def kernel(feat_user, feat_item, W_follows, b_follows, W_clicks, b_clicks,
           W_bought, b_bought, edge_index_follows, edge_index_clicks,
           edge_index_bought):
    def split_edges(ei):
        src = ei[0].reshape(NW, E_PER_W)
        dst = ei[1].reshape(NW, E_PER_W)
        src = jnp.pad(src, ((0, 0), (0, PAD_E)))
        dpad = DUMMY + (jnp.arange(PAD_E, dtype=jnp.int32) % 16)
        dst = jnp.concatenate(
            [dst, jnp.broadcast_to(dpad, (NW, PAD_E))], axis=1)
        return (src.reshape(NW, NCHUNKP, CHUNK), dst.reshape(NW, NCHUNKP, CHUNK))

    sf_, df_ = split_edges(edge_index_follows)
    sc_, dc_ = split_edges(edge_index_clicks)
    sb_, db_ = split_edges(edge_index_bought)

    z_s = jnp.zeros((RPT, D), jnp.float32)
    z_c = jnp.zeros((RPT, CNT_W), jnp.float32)
    sum_f, cnt_f, sum_c, cnt_c, sum_b, cnt_b = _sc_agg(
        feat_user, feat_item, z_s, z_c, sf_, df_, sc_, dc_, sb_, db_)

    h_user, h_item = _tc_finish(
        sum_f, cnt_f, sum_b, cnt_b, sum_c, cnt_c,
        W_follows, b_follows.reshape(1, D),
        W_bought, b_bought.reshape(1, D),
        W_clicks, b_clicks.reshape(1, D))
    return (h_user, h_item)


# speedup vs baseline: 1.1665x; 1.1665x over previous
import jax
import jax.numpy as jnp
from jax import lax
from jax.experimental import pallas as pl
from jax.experimental.pallas import tpu as pltpu
from jax.experimental.pallas import tpu_sc as plsc

N_USER = 10000
N_ITEM = 10000
E = 320000
D = 128
CNT_W = 16

NC, NS = 2, 16
NW = NC * NS
E_PER_W = E // NW
CHUNK = 80
NCHUNKP = 126
PAD_E = NCHUNKP * CHUNK - E_PER_W
NHALF = NCHUNKP // 2
DUMMY = N_USER
ACC_ROWS = N_USER + 16
RPT = 624
TAIL = N_USER - NS * RPT
TAIL0 = NS * RPT


def _sc_body(feat_user, feat_item, z_s, z_c, src_f, dst_f, src_c, dst_c,
             src_b, dst_b,
             out_sf, out_cf, out_sc, out_cc, out_sb, out_cb,
             acc_s, acc_c, sidx, didx, rows0, ones, g0, g1, s0, s1):
    c = lax.axis_index("c")
    s = lax.axis_index("s")
    wid = c * NS + s
    row0 = pl.multiple_of(s * RPT, 8)

    lane = lax.iota(jnp.int32, 16)
    one_vec = jnp.where(lane == 0, 1.0, 0.0).astype(jnp.float32)

    def fill_ones(i, carry):
        ones[i, :] = one_vec
        return carry

    lax.fori_loop(0, CHUNK, fill_ones, 0)

    def run_relation(feat, src_r, dst_r, out_s, out_c):
        d1 = pltpu.async_copy(z_s, acc_s.at[pl.ds(row0, RPT)], g0)
        d2 = pltpu.async_copy(z_c, acc_c.at[pl.ds(row0, RPT)], g1)
        d3 = pltpu.async_copy(src_r.at[wid], sidx, s0)
        d4 = pltpu.async_copy(dst_r.at[wid], didx, s1)

        @pl.when(s == 0)
        def _zero_tail():
            pltpu.async_copy(
                z_s.at[pl.ds(0, TAIL)], acc_s.at[pl.ds(TAIL0, TAIL)], g0).wait()
            pltpu.async_copy(
                z_c.at[pl.ds(0, TAIL)], acc_c.at[pl.ds(TAIL0, TAIL)], g1).wait()

        d1.wait()
        d2.wait()
        d3.wait()
        d4.wait()
        plsc.subcore_barrier()

        def g_issue(buf, j, sem):
            pltpu.async_copy(feat.at[sidx.at[j]], buf, sem)

        def g_wait(buf, sem):
            pltpu.make_async_copy(feat.at[sidx.at[0]], buf, sem).wait()

        def s_issue(buf, j, sem):
            pltpu.async_copy(buf, acc_s.at[didx.at[j]], sem, add=True)
            pltpu.async_copy(ones, acc_c.at[didx.at[j]], sem, add=True)

        def s_wait(buf, sem):
            pltpu.make_async_copy(buf, acc_s.at[didx.at[0]], sem).wait()
            pltpu.make_async_copy(ones, acc_c.at[didx.at[0]], sem).wait()

        def chunk(j, carry):
            pltpu.async_copy(feat.at[sidx.at[j]], rows0, g0).wait()
            pltpu.sync_copy(rows0, acc_s.at[didx.at[j]], add=True)
            pltpu.sync_copy(ones, acc_c.at[didx.at[j]], add=True)
            return carry

        lax.fori_loop(0, NCHUNKP, chunk, 0)

        plsc.subcore_barrier()
        pltpu.sync_copy(acc_s.at[pl.ds(row0, RPT)],
                        out_s.at[c, pl.ds(row0, RPT)])
        pltpu.sync_copy(acc_c.at[pl.ds(row0, RPT)],
                        out_c.at[c, pl.ds(row0, RPT)])

        @pl.when(s == 0)
        def _out_tail():
            pltpu.sync_copy(acc_s.at[pl.ds(TAIL0, TAIL)],
                            out_s.at[c, pl.ds(TAIL0, TAIL)])
            pltpu.sync_copy(acc_c.at[pl.ds(TAIL0, TAIL)],
                            out_c.at[c, pl.ds(TAIL0, TAIL)])

        plsc.subcore_barrier()

    run_relation(feat_user, src_f, dst_f, out_sf, out_cf)
    run_relation(feat_user, src_c, dst_c, out_sc, out_cc)
    run_relation(feat_item, src_b, dst_b, out_sb, out_cb)


_sc_agg = pl.kernel(
    _sc_body,
    out_type=[
        jax.ShapeDtypeStruct((NC, N_USER, D), jnp.float32),
        jax.ShapeDtypeStruct((NC, N_USER, CNT_W), jnp.float32),
        jax.ShapeDtypeStruct((NC, N_ITEM, D), jnp.float32),
        jax.ShapeDtypeStruct((NC, N_ITEM, CNT_W), jnp.float32),
        jax.ShapeDtypeStruct((NC, N_USER, D), jnp.float32),
        jax.ShapeDtypeStruct((NC, N_USER, CNT_W), jnp.float32),
    ],
    mesh=plsc.VectorSubcoreMesh(core_axis_name="c", subcore_axis_name="s"),
    compiler_params=pltpu.CompilerParams(use_tc_tiling_on_sc=False),
    scratch_types=[
        pltpu.VMEM_SHARED((ACC_ROWS, D), jnp.float32),
        pltpu.VMEM_SHARED((ACC_ROWS, CNT_W), jnp.float32),
        pltpu.VMEM((NCHUNKP, CHUNK), jnp.int32),
        pltpu.VMEM((NCHUNKP, CHUNK), jnp.int32),
        pltpu.VMEM((CHUNK, D), jnp.float32),
        pltpu.VMEM((CHUNK, CNT_W), jnp.float32),
        pltpu.SemaphoreType.DMA,
        pltpu.SemaphoreType.DMA,
        pltpu.SemaphoreType.DMA,
        pltpu.SemaphoreType.DMA,
    ],
)

BR = 400
NBLK = N_USER // BR


def _tc_body(sf, cf, sb, cb, sc_r, cc, wf, bf, wb, bb, wc, bc, hu, hi):
    def rel(sref, cref, wref, bref):
        tot = sref[0] + sref[1]
        cnt = cref[0, :, 0:1] + cref[1, :, 0:1]
        mean = tot / jnp.maximum(cnt, 1.0)
        h = lax.dot_general(mean, wref[...], (((1,), (1,)), ((), ())),
                            preferred_element_type=jnp.float32,
                            precision=lax.Precision.HIGHEST) + bref[...]
        return jnp.where(cnt > 0, h, 0.0)

    hu[...] = rel(sf, cf, wf, bf) + rel(sb, cb, wb, bb)
    hi[...] = rel(sc_r, cc, wc, bc)


_sum_spec = pl.BlockSpec((NC, BR, D), lambda i: (0, i, 0))
_cnt_spec = pl.BlockSpec((NC, BR, CNT_W), lambda i: (0, i, 0))
_w_spec = pl.BlockSpec((D, D), lambda i: (0, 0))
_b_spec = pl.BlockSpec((1, D), lambda i: (0, 0))

_tc_finish = pl.pallas_call(
    _tc_body,
    grid=(NBLK,),
    in_specs=[_sum_spec, _cnt_spec, _sum_spec, _cnt_spec, _sum_spec, _cnt_spec,
              _w_spec, _b_spec, _w_spec, _b_spec, _w_spec, _b_spec],
    out_specs=[pl.BlockSpec((BR, D), lambda i: (i, 0)),
               pl.BlockSpec((BR, D), lambda i: (i, 0))],
    out_shape=[jax.ShapeDtypeStruct((N_USER, D), jnp.float32),
               jax.ShapeDtypeStruct((N_ITEM, D), jnp.float32)],
)


def kernel(feat_user, feat_item, W_follows, b_follows, W_clicks, b_clicks,
           W_bought, b_bought, edge_index_follows, edge_index_clicks,
           edge_index_bought):
    def split_edges(ei):
        src = ei[0].reshape(NW, E_PER_W)
        dst = ei[1].reshape(NW, E_PER_W)
        src = jnp.pad(src, ((0, 0), (0, PAD_E)))
        dpad = DUMMY + (jnp.arange(PAD_E, dtype=jnp.int32) % 16)
        dst = jnp.concatenate(
            [dst, jnp.broadcast_to(dpad, (NW, PAD_E))], axis=1)
        return (src.reshape(NW, NCHUNKP, CHUNK), dst.reshape(NW, NCHUNKP, CHUNK))

    sf_, df_ = split_edges(edge_index_follows)
    sc_, dc_ = split_edges(edge_index_clicks)
    sb_, db_ = split_edges(edge_index_bought)

    z_s = jnp.zeros((RPT, D), jnp.float32)
    z_c = jnp.zeros((RPT, CNT_W), jnp.float32)
    sum_f, cnt_f, sum_c, cnt_c, sum_b, cnt_b = _sc_agg(
        feat_user, feat_item, z_s, z_c, sf_, df_, sc_, dc_, sb_, db_)

    h_user, h_item = _tc_finish(
        sum_f, cnt_f, sum_b, cnt_b, sum_c, cnt_c,
        W_follows, b_follows.reshape(1, D),
        W_bought, b_bought.reshape(1, D),
        W_clicks, b_clicks.reshape(1, D))
    return (h_user, h_item)


# speedup vs baseline: 1.5203x; 1.3033x over previous
import jax
import jax.numpy as jnp
from jax import lax
from jax.experimental import pallas as pl
from jax.experimental.pallas import tpu as pltpu
from jax.experimental.pallas import tpu_sc as plsc

N_USER = 10000
N_ITEM = 10000
E = 320000
D = 128
CNT_W = 16

NC, NS = 2, 16
NW = NC * NS
E_PER_W = E // NW
CHUNK = 80
NCHUNKP = 126
PAD_E = NCHUNKP * CHUNK - E_PER_W
NHALF = NCHUNKP // 2
DUMMY = N_USER
ACC_ROWS = N_USER + 16
RPT = 624
TAIL = N_USER - NS * RPT
TAIL0 = NS * RPT


def _sc_body(feat_user, feat_item, z_s, z_c, src_f, dst_f, src_c, dst_c,
             src_b, dst_b,
             out_sf, out_cf, out_sc, out_cc, out_sb, out_cb,
             acc_s, acc_c, sidx, didx, rows0, ones, g0, g1, s0, s1):
    c = lax.axis_index("c")
    s = lax.axis_index("s")
    wid = c * NS + s
    row0 = pl.multiple_of(s * RPT, 8)

    lane = lax.iota(jnp.int32, 16)
    one_vec = jnp.where(lane == 0, 1.0, 0.0).astype(jnp.float32)

    def fill_ones(i, carry):
        ones[i, :] = one_vec
        return carry

    lax.fori_loop(0, CHUNK, fill_ones, 0)

    def run_relation(feat, src_r, dst_r, out_s, out_c):
        d1 = pltpu.async_copy(z_s, acc_s.at[pl.ds(row0, RPT)], g0)
        d2 = pltpu.async_copy(z_c, acc_c.at[pl.ds(row0, RPT)], g1)
        d3 = pltpu.async_copy(src_r.at[wid], sidx, s0)
        d4 = pltpu.async_copy(dst_r.at[wid], didx, s1)

        @pl.when(s == 0)
        def _zero_tail():
            pltpu.async_copy(
                z_s.at[pl.ds(0, TAIL)], acc_s.at[pl.ds(TAIL0, TAIL)], g0).wait()
            pltpu.async_copy(
                z_c.at[pl.ds(0, TAIL)], acc_c.at[pl.ds(TAIL0, TAIL)], g1).wait()

        d1.wait()
        d2.wait()
        d3.wait()
        d4.wait()
        plsc.subcore_barrier()

        def g_issue(buf, j, sem):
            pltpu.async_copy(feat.at[sidx.at[j]], buf, sem)

        def g_wait(buf, sem):
            pltpu.make_async_copy(feat.at[sidx.at[0]], buf, sem).wait()

        def s_issue(buf, j, sem):
            pltpu.async_copy(buf, acc_s.at[didx.at[j]], sem, add=True)
            pltpu.async_copy(ones, acc_c.at[didx.at[j]], sem, add=True)

        def s_wait(buf, sem):
            pltpu.make_async_copy(buf, acc_s.at[didx.at[0]], sem).wait()
            pltpu.make_async_copy(ones, acc_c.at[didx.at[0]], sem).wait()

        def chunk(j, carry):
            pltpu.async_copy(feat.at[sidx.at[j]], rows0, g0).wait()
            pltpu.sync_copy(rows0, acc_s.at[didx.at[j]], add=True)
            pltpu.sync_copy(ones, acc_c.at[didx.at[j]], add=True)
            return carry

        lax.fori_loop(0, NCHUNKP, chunk, 0)

        plsc.subcore_barrier()
        pltpu.sync_copy(acc_s.at[pl.ds(row0, RPT)],
                        out_s.at[c, pl.ds(row0, RPT)])
        pltpu.sync_copy(acc_c.at[pl.ds(row0, RPT)],
                        out_c.at[c, pl.ds(row0, RPT)])

        @pl.when(s == 0)
        def _out_tail():
            pltpu.sync_copy(acc_s.at[pl.ds(TAIL0, TAIL)],
                            out_s.at[c, pl.ds(TAIL0, TAIL)])
            pltpu.sync_copy(acc_c.at[pl.ds(TAIL0, TAIL)],
                            out_c.at[c, pl.ds(TAIL0, TAIL)])

        plsc.subcore_barrier()

    run_relation(feat_user, src_f, dst_f, out_sf, out_cf)
    run_relation(feat_user, src_c, dst_c, out_sc, out_cc)
    run_relation(feat_item, src_b, dst_b, out_sb, out_cb)


_sc_agg = pl.kernel(
    _sc_body,
    out_type=[
        jax.ShapeDtypeStruct((NC, N_USER, D), jnp.float32),
        jax.ShapeDtypeStruct((NC, N_USER, CNT_W), jnp.float32),
        jax.ShapeDtypeStruct((NC, N_ITEM, D), jnp.float32),
        jax.ShapeDtypeStruct((NC, N_ITEM, CNT_W), jnp.float32),
        jax.ShapeDtypeStruct((NC, N_USER, D), jnp.float32),
        jax.ShapeDtypeStruct((NC, N_USER, CNT_W), jnp.float32),
    ],
    mesh=plsc.VectorSubcoreMesh(core_axis_name="c", subcore_axis_name="s"),
    compiler_params=pltpu.CompilerParams(use_tc_tiling_on_sc=False),
    scratch_types=[
        pltpu.VMEM_SHARED((ACC_ROWS, D), jnp.float32),
        pltpu.VMEM_SHARED((ACC_ROWS, CNT_W), jnp.float32),
        pltpu.VMEM((NCHUNKP, CHUNK), jnp.int32),
        pltpu.VMEM((NCHUNKP, CHUNK), jnp.int32),
        pltpu.VMEM((CHUNK, D), jnp.float32),
        pltpu.VMEM((CHUNK, CNT_W), jnp.float32),
        pltpu.SemaphoreType.DMA,
        pltpu.SemaphoreType.DMA,
        pltpu.SemaphoreType.DMA,
        pltpu.SemaphoreType.DMA,
    ],
)

BR = 400
NBLK = N_USER // BR


def _tc_body(sf, cf, sb, cb, sc_r, cc, wf, bf, wb, bb, wc, bc, hu, hi):
    def rel(sref, cref, wref, bref):
        tot = sref[0] + sref[1]
        cnt = cref[0, :, 0:1] + cref[1, :, 0:1]
        mean = tot / jnp.maximum(cnt, 1.0)
        h = lax.dot_general(mean, wref[...], (((1,), (1,)), ((), ())),
                            preferred_element_type=jnp.float32,
                            precision=lax.Precision.HIGHEST) + bref[...]
        return jnp.where(cnt > 0, h, 0.0)

    hu[...] = rel(sf, cf, wf, bf) + rel(sb, cb, wb, bb)
    hi[...] = rel(sc_r, cc, wc, bc)


_sum_spec = pl.BlockSpec((NC, BR, D), lambda i: (0, i, 0))
_cnt_spec = pl.BlockSpec((NC, BR, CNT_W), lambda i: (0, i, 0))
_w_spec = pl.BlockSpec((D, D), lambda i: (0, 0))
_b_spec = pl.BlockSpec((1, D), lambda i: (0, 0))

_tc_finish = pl.pallas_call(
    _tc_body,
    grid=(NBLK,),
    in_specs=[_sum_spec, _cnt_spec, _sum_spec, _cnt_spec, _sum_spec, _cnt_spec,
              _w_spec, _b_spec, _w_spec, _b_spec, _w_spec, _b_spec],
    out_specs=[pl.BlockSpec((BR, D), lambda i: (i, 0)),
               pl.BlockSpec((BR, D), lambda i: (i, 0))],
    out_shape=[jax.ShapeDtypeStruct((N_USER, D), jnp.float32),
               jax.ShapeDtypeStruct((N_ITEM, D), jnp.float32)],
)


def kernel(feat_user, feat_item, W_follows, b_follows, W_clicks, b_clicks,
           W_bought, b_bought, edge_index_follows, edge_index_clicks,
           edge_index_bought):
    def split_edges(ei):
        src = ei[0].reshape(NW, E_PER_W)
        dst = ei[1].reshape(NW, E_PER_W)
        spad = jnp.arange(PAD_E, dtype=jnp.int32) * 97 % N_USER
        dpad = DUMMY + (jnp.arange(PAD_E, dtype=jnp.int32) % 16)
        src = jnp.concatenate(
            [src, jnp.broadcast_to(spad, (NW, PAD_E))], axis=1)
        dst = jnp.concatenate(
            [dst, jnp.broadcast_to(dpad, (NW, PAD_E))], axis=1)
        return (src.reshape(NW, NCHUNKP, CHUNK), dst.reshape(NW, NCHUNKP, CHUNK))

    sf_, df_ = split_edges(edge_index_follows)
    sc_, dc_ = split_edges(edge_index_clicks)
    sb_, db_ = split_edges(edge_index_bought)

    z_s = jnp.zeros((RPT, D), jnp.float32)
    z_c = jnp.zeros((RPT, CNT_W), jnp.float32)
    sum_f, cnt_f, sum_c, cnt_c, sum_b, cnt_b = _sc_agg(
        feat_user, feat_item, z_s, z_c, sf_, df_, sc_, dc_, sb_, db_)

    h_user, h_item = _tc_finish(
        sum_f, cnt_f, sum_b, cnt_b, sum_c, cnt_c,
        W_follows, b_follows.reshape(1, D),
        W_bought, b_bought.reshape(1, D),
        W_clicks, b_clicks.reshape(1, D))
    return (h_user, h_item)


# speedup vs baseline: 1.9287x; 1.2687x over previous
import jax
import jax.numpy as jnp
from jax import lax
from jax.experimental import pallas as pl
from jax.experimental.pallas import tpu as pltpu
from jax.experimental.pallas import tpu_sc as plsc

N_USER = 10000
N_ITEM = 10000
E = 320000
D = 128
CNT_W = 16

NC, NS = 2, 16
NW = NC * NS
E_PER_W = E // NW
CHUNK = 72
NCHUNKP = 140
PAD_E = NCHUNKP * CHUNK - E_PER_W
NHALF = NCHUNKP // 2
DUMMY = N_USER
ACC_ROWS = N_USER + 16
RPT = 624
TAIL = N_USER - NS * RPT
TAIL0 = NS * RPT


def _sc_body(feat_user, feat_item, z_s, z_c, src_f, dst_f, src_c, dst_c,
             src_b, dst_b,
             out_sf, out_cf, out_sc, out_cc, out_sb, out_cb,
             acc_s, acc_c, sidx, didx, rows0, rows1, ones, g0, g1, s0, s1):
    c = lax.axis_index("c")
    s = lax.axis_index("s")
    wid = c * NS + s
    row0 = pl.multiple_of(s * RPT, 8)

    lane = lax.iota(jnp.int32, 16)
    one_vec = jnp.where(lane == 0, 1.0, 0.0).astype(jnp.float32)

    def fill_ones(i, carry):
        ones[i, :] = one_vec
        return carry

    lax.fori_loop(0, CHUNK, fill_ones, 0)

    def run_relation(feat, src_r, dst_r, out_s, out_c):
        d1 = pltpu.async_copy(z_s, acc_s.at[pl.ds(row0, RPT)], g0)
        d2 = pltpu.async_copy(z_c, acc_c.at[pl.ds(row0, RPT)], g1)
        d3 = pltpu.async_copy(src_r.at[wid], sidx, s0)
        d4 = pltpu.async_copy(dst_r.at[wid], didx, s1)

        @pl.when(s == 0)
        def _zero_tail():
            pltpu.async_copy(
                z_s.at[pl.ds(0, TAIL)], acc_s.at[pl.ds(TAIL0, TAIL)], g0).wait()
            pltpu.async_copy(
                z_c.at[pl.ds(0, TAIL)], acc_c.at[pl.ds(TAIL0, TAIL)], g1).wait()

        d1.wait()
        d2.wait()
        d3.wait()
        d4.wait()
        plsc.subcore_barrier()

        def g_issue(buf, j, sem):
            pltpu.async_copy(feat.at[sidx.at[j]], buf, sem)

        def g_wait(buf, sem):
            pltpu.make_async_copy(feat.at[sidx.at[0]], buf, sem).wait()

        def s_issue(buf, j, sem):
            pltpu.async_copy(buf, acc_s.at[didx.at[j]], sem, add=True)
            pltpu.async_copy(ones, acc_c.at[didx.at[j]], sem, add=True)

        def s_wait(buf, sem):
            pltpu.make_async_copy(buf, acc_s.at[didx.at[0]], sem).wait()
            pltpu.make_async_copy(ones, acc_c.at[didx.at[0]], sem).wait()

        g_issue(rows0, 0, g0)
        g_issue(rows1, 1, g1)

        def step(i, carry):
            a = 2 * i
            g_wait(rows0, g0)
            s_issue(rows0, a, s0)
            g_wait(rows1, g1)
            s_issue(rows1, a + 1, s1)
            s_wait(rows0, s0)
            g_issue(rows0, a + 2, g0)
            s_wait(rows1, s1)
            g_issue(rows1, a + 3, g1)
            return carry

        lax.fori_loop(0, NHALF - 1, step, 0)
        g_wait(rows0, g0)
        s_issue(rows0, NCHUNKP - 2, s0)
        g_wait(rows1, g1)
        s_issue(rows1, NCHUNKP - 1, s1)
        s_wait(rows0, s0)
        s_wait(rows1, s1)

        plsc.subcore_barrier()
        pltpu.sync_copy(acc_s.at[pl.ds(row0, RPT)],
                        out_s.at[c, pl.ds(row0, RPT)])
        pltpu.sync_copy(acc_c.at[pl.ds(row0, RPT)],
                        out_c.at[c, pl.ds(row0, RPT)])

        @pl.when(s == 0)
        def _out_tail():
            pltpu.sync_copy(acc_s.at[pl.ds(TAIL0, TAIL)],
                            out_s.at[c, pl.ds(TAIL0, TAIL)])
            pltpu.sync_copy(acc_c.at[pl.ds(TAIL0, TAIL)],
                            out_c.at[c, pl.ds(TAIL0, TAIL)])

        plsc.subcore_barrier()

    run_relation(feat_user, src_f, dst_f, out_sf, out_cf)
    run_relation(feat_user, src_c, dst_c, out_sc, out_cc)
    run_relation(feat_item, src_b, dst_b, out_sb, out_cb)


_sc_agg = pl.kernel(
    _sc_body,
    out_type=[
        jax.ShapeDtypeStruct((NC, N_USER, D), jnp.float32),
        jax.ShapeDtypeStruct((NC, N_USER, CNT_W), jnp.float32),
        jax.ShapeDtypeStruct((NC, N_ITEM, D), jnp.float32),
        jax.ShapeDtypeStruct((NC, N_ITEM, CNT_W), jnp.float32),
        jax.ShapeDtypeStruct((NC, N_USER, D), jnp.float32),
        jax.ShapeDtypeStruct((NC, N_USER, CNT_W), jnp.float32),
    ],
    mesh=plsc.VectorSubcoreMesh(core_axis_name="c", subcore_axis_name="s"),
    compiler_params=pltpu.CompilerParams(use_tc_tiling_on_sc=False),
    scratch_types=[
        pltpu.VMEM_SHARED((ACC_ROWS, D), jnp.float32),
        pltpu.VMEM_SHARED((ACC_ROWS, CNT_W), jnp.float32),
        pltpu.VMEM((NCHUNKP, CHUNK), jnp.int32),
        pltpu.VMEM((NCHUNKP, CHUNK), jnp.int32),
        pltpu.VMEM((CHUNK, D), jnp.float32),
        pltpu.VMEM((CHUNK, D), jnp.float32),
        pltpu.VMEM((CHUNK, CNT_W), jnp.float32),
        pltpu.SemaphoreType.DMA,
        pltpu.SemaphoreType.DMA,
        pltpu.SemaphoreType.DMA,
        pltpu.SemaphoreType.DMA,
    ],
)

BR = 400
NBLK = N_USER // BR


def _tc_body(sf, cf, sb, cb, sc_r, cc, wf, bf, wb, bb, wc, bc, hu, hi):
    def rel(sref, cref, wref, bref):
        tot = sref[0] + sref[1]
        cnt = cref[0, :, 0:1] + cref[1, :, 0:1]
        mean = tot / jnp.maximum(cnt, 1.0)
        h = lax.dot_general(mean, wref[...], (((1,), (1,)), ((), ())),
                            preferred_element_type=jnp.float32,
                            precision=lax.Precision.HIGHEST) + bref[...]
        return jnp.where(cnt > 0, h, 0.0)

    hu[...] = rel(sf, cf, wf, bf) + rel(sb, cb, wb, bb)
    hi[...] = rel(sc_r, cc, wc, bc)


_sum_spec = pl.BlockSpec((NC, BR, D), lambda i: (0, i, 0))
_cnt_spec = pl.BlockSpec((NC, BR, CNT_W), lambda i: (0, i, 0))
_w_spec = pl.BlockSpec((D, D), lambda i: (0, 0))
_b_spec = pl.BlockSpec((1, D), lambda i: (0, 0))

_tc_finish = pl.pallas_call(
    _tc_body,
    grid=(NBLK,),
    in_specs=[_sum_spec, _cnt_spec, _sum_spec, _cnt_spec, _sum_spec, _cnt_spec,
              _w_spec, _b_spec, _w_spec, _b_spec, _w_spec, _b_spec],
    out_specs=[pl.BlockSpec((BR, D), lambda i: (i, 0)),
               pl.BlockSpec((BR, D), lambda i: (i, 0))],
    out_shape=[jax.ShapeDtypeStruct((N_USER, D), jnp.float32),
               jax.ShapeDtypeStruct((N_ITEM, D), jnp.float32)],
)


def kernel(feat_user, feat_item, W_follows, b_follows, W_clicks, b_clicks,
           W_bought, b_bought, edge_index_follows, edge_index_clicks,
           edge_index_bought):
    def split_edges(ei):
        src = ei[0].reshape(NW, E_PER_W)
        dst = ei[1].reshape(NW, E_PER_W)
        spad = jnp.arange(PAD_E, dtype=jnp.int32) * 97 % N_USER
        dpad = DUMMY + (jnp.arange(PAD_E, dtype=jnp.int32) % 16)
        src = jnp.concatenate(
            [src, jnp.broadcast_to(spad, (NW, PAD_E))], axis=1)
        dst = jnp.concatenate(
            [dst, jnp.broadcast_to(dpad, (NW, PAD_E))], axis=1)
        return (src.reshape(NW, NCHUNKP, CHUNK), dst.reshape(NW, NCHUNKP, CHUNK))

    sf_, df_ = split_edges(edge_index_follows)
    sc_, dc_ = split_edges(edge_index_clicks)
    sb_, db_ = split_edges(edge_index_bought)

    z_s = jnp.zeros((RPT, D), jnp.float32)
    z_c = jnp.zeros((RPT, CNT_W), jnp.float32)
    sum_f, cnt_f, sum_c, cnt_c, sum_b, cnt_b = _sc_agg(
        feat_user, feat_item, z_s, z_c, sf_, df_, sc_, dc_, sb_, db_)

    h_user, h_item = _tc_finish(
        sum_f, cnt_f, sum_b, cnt_b, sum_c, cnt_c,
        W_follows, b_follows.reshape(1, D),
        W_bought, b_bought.reshape(1, D),
        W_clicks, b_clicks.reshape(1, D))
    return (h_user, h_item)


# speedup vs baseline: 1.9349x; 1.0032x over previous
import jax
import jax.numpy as jnp
from jax import lax
from jax.experimental import pallas as pl
from jax.experimental.pallas import tpu as pltpu
from jax.experimental.pallas import tpu_sc as plsc

N_USER = 10000
N_ITEM = 10000
E = 320000
D = 128
CNT_W = 16

NC, NS = 2, 16
NW = NC * NS
E_PER_W = E // NW
CHUNK = 72
NCHUNKP = 140
PAD_E = NCHUNKP * CHUNK - E_PER_W
NHALF = NCHUNKP // 2
DUMMY = N_USER
ACC_ROWS = N_USER + 16
RPT = 624
TAIL = N_USER - NS * RPT
TAIL0 = NS * RPT


def _sc_body(feat_user, feat_item, z_s, z_c, src_f, dst_f, src_c, dst_c,
             src_b, dst_b,
             out_sf, out_cf, out_sc, out_cc, out_sb, out_cb,
             acc_s, acc_c, sidx, didx, rows0, rows1, ones, g0, g1, s0, s1):
    c = lax.axis_index("c")
    s = lax.axis_index("s")
    wid = c * NS + s
    row0 = pl.multiple_of(s * RPT, 8)

    lane = lax.iota(jnp.int32, 16)
    one_vec = jnp.where(lane == 0, 1.0, 0.0).astype(jnp.float32)

    def fill_ones(i, carry):
        ones[i, :] = one_vec
        return carry

    lax.fori_loop(0, CHUNK, fill_ones, 0)

    def run_relation(feat, src_r, dst_r, out_s, out_c):
        d1 = pltpu.async_copy(z_s, acc_s.at[pl.ds(row0, RPT)], g0)
        d2 = pltpu.async_copy(z_c, acc_c.at[pl.ds(row0, RPT)], g1)
        d3 = pltpu.async_copy(src_r.at[wid], sidx, s0)
        d4 = pltpu.async_copy(dst_r.at[wid], didx, s1)

        @pl.when(s == 0)
        def _zero_tail():
            pltpu.async_copy(
                z_s.at[pl.ds(0, TAIL)], acc_s.at[pl.ds(TAIL0, TAIL)], g0).wait()
            pltpu.async_copy(
                z_c.at[pl.ds(0, TAIL)], acc_c.at[pl.ds(TAIL0, TAIL)], g1).wait()

        d1.wait()
        d2.wait()
        d3.wait()
        d4.wait()
        plsc.subcore_barrier()

        def g_issue(buf, j, sem):
            pltpu.async_copy(feat.at[sidx.at[j]], buf, sem)

        def g_wait(buf, sem):
            pltpu.make_async_copy(feat.at[sidx.at[0]], buf, sem).wait()

        def s_issue(buf, j, sem):
            pltpu.async_copy(buf, acc_s.at[didx.at[j]], sem, add=True)
            pltpu.async_copy(ones, acc_c.at[didx.at[j]], sem, add=True)

        def s_wait(buf, sem):
            pltpu.make_async_copy(buf, acc_s.at[didx.at[0]], sem).wait()
            pltpu.make_async_copy(ones, acc_c.at[didx.at[0]], sem).wait()

        g_issue(rows0, 0, g0)
        g_issue(rows1, 1, g1)

        def step(i, carry):
            a = 2 * i
            g_wait(rows0, g0)
            s_issue(rows0, a, s0)
            g_wait(rows1, g1)
            s_issue(rows1, a + 1, s1)
            s_wait(rows0, s0)
            g_issue(rows0, a + 2, g0)
            s_wait(rows1, s1)
            g_issue(rows1, a + 3, g1)
            return carry

        lax.fori_loop(0, NHALF - 1, step, 0)
        g_wait(rows0, g0)
        s_issue(rows0, NCHUNKP - 2, s0)
        g_wait(rows1, g1)
        s_issue(rows1, NCHUNKP - 1, s1)
        s_wait(rows0, s0)
        s_wait(rows1, s1)

        plsc.subcore_barrier()
        o1 = pltpu.async_copy(acc_s.at[pl.ds(row0, RPT)],
                              out_s.at[c, pl.ds(row0, RPT)], g0)
        o2 = pltpu.async_copy(acc_c.at[pl.ds(row0, RPT)],
                              out_c.at[c, pl.ds(row0, RPT)], g1)

        @pl.when(s == 0)
        def _out_tail():
            pltpu.async_copy(acc_s.at[pl.ds(TAIL0, TAIL)],
                             out_s.at[c, pl.ds(TAIL0, TAIL)], s0).wait()
            pltpu.async_copy(acc_c.at[pl.ds(TAIL0, TAIL)],
                             out_c.at[c, pl.ds(TAIL0, TAIL)], s1).wait()

        o1.wait()
        o2.wait()
        plsc.subcore_barrier()

    run_relation(feat_user, src_f, dst_f, out_sf, out_cf)
    run_relation(feat_user, src_c, dst_c, out_sc, out_cc)
    run_relation(feat_item, src_b, dst_b, out_sb, out_cb)


_sc_agg = pl.kernel(
    _sc_body,
    out_type=[
        jax.ShapeDtypeStruct((NC, N_USER, D), jnp.float32),
        jax.ShapeDtypeStruct((NC, N_USER, CNT_W), jnp.float32),
        jax.ShapeDtypeStruct((NC, N_ITEM, D), jnp.float32),
        jax.ShapeDtypeStruct((NC, N_ITEM, CNT_W), jnp.float32),
        jax.ShapeDtypeStruct((NC, N_USER, D), jnp.float32),
        jax.ShapeDtypeStruct((NC, N_USER, CNT_W), jnp.float32),
    ],
    mesh=plsc.VectorSubcoreMesh(core_axis_name="c", subcore_axis_name="s"),
    compiler_params=pltpu.CompilerParams(use_tc_tiling_on_sc=False),
    scratch_types=[
        pltpu.VMEM_SHARED((ACC_ROWS, D), jnp.float32),
        pltpu.VMEM_SHARED((ACC_ROWS, CNT_W), jnp.float32),
        pltpu.VMEM((NCHUNKP, CHUNK), jnp.int32),
        pltpu.VMEM((NCHUNKP, CHUNK), jnp.int32),
        pltpu.VMEM((CHUNK, D), jnp.float32),
        pltpu.VMEM((CHUNK, D), jnp.float32),
        pltpu.VMEM((CHUNK, CNT_W), jnp.float32),
        pltpu.SemaphoreType.DMA,
        pltpu.SemaphoreType.DMA,
        pltpu.SemaphoreType.DMA,
        pltpu.SemaphoreType.DMA,
    ],
)

BR = 400
NBLK = N_USER // BR


def _tc_body(sf, cf, sb, cb, sc_r, cc, wf, bf, wb, bb, wc, bc, hu, hi):
    def rel(sref, cref, wref, bref):
        tot = sref[0] + sref[1]
        cnt = cref[0, :, 0:1] + cref[1, :, 0:1]
        mean = tot / jnp.maximum(cnt, 1.0)
        h = lax.dot_general(mean, wref[...], (((1,), (1,)), ((), ())),
                            preferred_element_type=jnp.float32,
                            precision=lax.Precision.HIGHEST) + bref[...]
        return jnp.where(cnt > 0, h, 0.0)

    hu[...] = rel(sf, cf, wf, bf) + rel(sb, cb, wb, bb)
    hi[...] = rel(sc_r, cc, wc, bc)


_sum_spec = pl.BlockSpec((NC, BR, D), lambda i: (0, i, 0))
_cnt_spec = pl.BlockSpec((NC, BR, CNT_W), lambda i: (0, i, 0))
_w_spec = pl.BlockSpec((D, D), lambda i: (0, 0))
_b_spec = pl.BlockSpec((1, D), lambda i: (0, 0))

_tc_finish = pl.pallas_call(
    _tc_body,
    grid=(NBLK,),
    in_specs=[_sum_spec, _cnt_spec, _sum_spec, _cnt_spec, _sum_spec, _cnt_spec,
              _w_spec, _b_spec, _w_spec, _b_spec, _w_spec, _b_spec],
    out_specs=[pl.BlockSpec((BR, D), lambda i: (i, 0)),
               pl.BlockSpec((BR, D), lambda i: (i, 0))],
    out_shape=[jax.ShapeDtypeStruct((N_USER, D), jnp.float32),
               jax.ShapeDtypeStruct((N_ITEM, D), jnp.float32)],
)


def kernel(feat_user, feat_item, W_follows, b_follows, W_clicks, b_clicks,
           W_bought, b_bought, edge_index_follows, edge_index_clicks,
           edge_index_bought):
    def split_edges(ei):
        src = ei[0].reshape(NW, E_PER_W)
        dst = ei[1].reshape(NW, E_PER_W)
        spad = jnp.arange(PAD_E, dtype=jnp.int32) * 97 % N_USER
        dpad = DUMMY + (jnp.arange(PAD_E, dtype=jnp.int32) % 16)
        src = jnp.concatenate(
            [src, jnp.broadcast_to(spad, (NW, PAD_E))], axis=1)
        dst = jnp.concatenate(
            [dst, jnp.broadcast_to(dpad, (NW, PAD_E))], axis=1)
        return (src.reshape(NW, NCHUNKP, CHUNK), dst.reshape(NW, NCHUNKP, CHUNK))

    sf_, df_ = split_edges(edge_index_follows)
    sc_, dc_ = split_edges(edge_index_clicks)
    sb_, db_ = split_edges(edge_index_bought)

    z_s = jnp.zeros((RPT, D), jnp.float32)
    z_c = jnp.zeros((RPT, CNT_W), jnp.float32)
    sum_f, cnt_f, sum_c, cnt_c, sum_b, cnt_b = _sc_agg(
        feat_user, feat_item, z_s, z_c, sf_, df_, sc_, dc_, sb_, db_)

    h_user, h_item = _tc_finish(
        sum_f, cnt_f, sum_b, cnt_b, sum_c, cnt_c,
        W_follows, b_follows.reshape(1, D),
        W_bought, b_bought.reshape(1, D),
        W_clicks, b_clicks.reshape(1, D))
    return (h_user, h_item)


# speedup vs baseline: 1.9429x; 1.0041x over previous
import jax
import jax.numpy as jnp
from jax import lax
from jax.experimental import pallas as pl
from jax.experimental.pallas import tpu as pltpu
from jax.experimental.pallas import tpu_sc as plsc

N_USER = 10000
N_ITEM = 10000
E = 320000
D = 128
CNT_W = 8

NC, NS = 2, 16
NW = NC * NS
E_PER_W = E // NW
CHUNK = 72
NCHUNKP = 140
PAD_E = NCHUNKP * CHUNK - E_PER_W
NHALF = NCHUNKP // 2
DUMMY = N_USER
ACC_ROWS = N_USER + 16
RPT = 624
TAIL = N_USER - NS * RPT
TAIL0 = NS * RPT


def _sc_body(feat_user, feat_item, z_s, z_c, ones_h, src_f, dst_f, src_c, dst_c,
             src_b, dst_b,
             out_sf, out_cf, out_sc, out_cc, out_sb, out_cb,
             acc_s, acc_c, sidx, didx, rows0, rows1, ones, g0, g1, s0, s1):
    c = lax.axis_index("c")
    s = lax.axis_index("s")
    wid = c * NS + s
    row0 = pl.multiple_of(s * RPT, 8)

    pltpu.sync_copy(ones_h, ones)

    def run_relation(feat, src_r, dst_r, out_s, out_c):
        d1 = pltpu.async_copy(z_s, acc_s.at[pl.ds(row0, RPT)], g0)
        d2 = pltpu.async_copy(z_c, acc_c.at[pl.ds(row0, RPT)], g1)
        d3 = pltpu.async_copy(src_r.at[wid], sidx, s0)
        d4 = pltpu.async_copy(dst_r.at[wid], didx, s1)

        @pl.when(s == 0)
        def _zero_tail():
            pltpu.async_copy(
                z_s.at[pl.ds(0, TAIL)], acc_s.at[pl.ds(TAIL0, TAIL)], g0).wait()
            pltpu.async_copy(
                z_c.at[pl.ds(0, TAIL)], acc_c.at[pl.ds(TAIL0, TAIL)], g1).wait()

        d1.wait()
        d2.wait()
        d3.wait()
        d4.wait()
        plsc.subcore_barrier()

        def g_issue(buf, j, sem):
            pltpu.async_copy(feat.at[sidx.at[j]], buf, sem)

        def g_wait(buf, sem):
            pltpu.make_async_copy(feat.at[sidx.at[0]], buf, sem).wait()

        def s_issue(buf, j, sem):
            pltpu.async_copy(buf, acc_s.at[didx.at[j]], sem, add=True)
            pltpu.async_copy(ones, acc_c.at[didx.at[j]], sem, add=True)

        def s_wait(buf, sem):
            pltpu.make_async_copy(buf, acc_s.at[didx.at[0]], sem).wait()
            pltpu.make_async_copy(ones, acc_c.at[didx.at[0]], sem).wait()

        g_issue(rows0, 0, g0)
        g_issue(rows1, 1, g1)

        def step(i, carry):
            a = 2 * i
            g_wait(rows0, g0)
            s_issue(rows0, a, s0)
            g_wait(rows1, g1)
            s_issue(rows1, a + 1, s1)
            s_wait(rows0, s0)
            g_issue(rows0, a + 2, g0)
            s_wait(rows1, s1)
            g_issue(rows1, a + 3, g1)
            return carry

        lax.fori_loop(0, NHALF - 1, step, 0)
        g_wait(rows0, g0)
        s_issue(rows0, NCHUNKP - 2, s0)
        g_wait(rows1, g1)
        s_issue(rows1, NCHUNKP - 1, s1)
        s_wait(rows0, s0)
        s_wait(rows1, s1)

        plsc.subcore_barrier()
        o1 = pltpu.async_copy(acc_s.at[pl.ds(row0, RPT)],
                              out_s.at[c, pl.ds(row0, RPT)], g0)
        o2 = pltpu.async_copy(acc_c.at[pl.ds(row0, RPT)],
                              out_c.at[c, pl.ds(row0, RPT)], g1)

        @pl.when(s == 0)
        def _out_tail():
            pltpu.async_copy(acc_s.at[pl.ds(TAIL0, TAIL)],
                             out_s.at[c, pl.ds(TAIL0, TAIL)], s0).wait()
            pltpu.async_copy(acc_c.at[pl.ds(TAIL0, TAIL)],
                             out_c.at[c, pl.ds(TAIL0, TAIL)], s1).wait()

        o1.wait()
        o2.wait()
        plsc.subcore_barrier()

    run_relation(feat_user, src_f, dst_f, out_sf, out_cf)
    run_relation(feat_user, src_c, dst_c, out_sc, out_cc)
    run_relation(feat_item, src_b, dst_b, out_sb, out_cb)


_sc_agg = pl.kernel(
    _sc_body,
    out_type=[
        jax.ShapeDtypeStruct((NC, N_USER, D), jnp.float32),
        jax.ShapeDtypeStruct((NC, N_USER, CNT_W), jnp.float32),
        jax.ShapeDtypeStruct((NC, N_ITEM, D), jnp.float32),
        jax.ShapeDtypeStruct((NC, N_ITEM, CNT_W), jnp.float32),
        jax.ShapeDtypeStruct((NC, N_USER, D), jnp.float32),
        jax.ShapeDtypeStruct((NC, N_USER, CNT_W), jnp.float32),
    ],
    mesh=plsc.VectorSubcoreMesh(core_axis_name="c", subcore_axis_name="s"),
    compiler_params=pltpu.CompilerParams(use_tc_tiling_on_sc=False),
    scratch_types=[
        pltpu.VMEM_SHARED((ACC_ROWS, D), jnp.float32),
        pltpu.VMEM_SHARED((ACC_ROWS, CNT_W), jnp.float32),
        pltpu.VMEM((NCHUNKP, CHUNK), jnp.int32),
        pltpu.VMEM((NCHUNKP, CHUNK), jnp.int32),
        pltpu.VMEM((CHUNK, D), jnp.float32),
        pltpu.VMEM((CHUNK, D), jnp.float32),
        pltpu.VMEM((CHUNK, CNT_W), jnp.float32),
        pltpu.SemaphoreType.DMA,
        pltpu.SemaphoreType.DMA,
        pltpu.SemaphoreType.DMA,
        pltpu.SemaphoreType.DMA,
    ],
)

BR = 400
NBLK = N_USER // BR


def _tc_body(sf, cf, sb, cb, sc_r, cc, wf, bf, wb, bb, wc, bc, hu, hi):
    def rel(sref, cref, wref, bref):
        tot = sref[0] + sref[1]
        cnt = cref[0, :, 0:1] + cref[1, :, 0:1]
        mean = tot / jnp.maximum(cnt, 1.0)
        h = lax.dot_general(mean, wref[...], (((1,), (1,)), ((), ())),
                            preferred_element_type=jnp.float32,
                            precision=lax.Precision.HIGHEST) + bref[...]
        return jnp.where(cnt > 0, h, 0.0)

    hu[...] = rel(sf, cf, wf, bf) + rel(sb, cb, wb, bb)
    hi[...] = rel(sc_r, cc, wc, bc)


_sum_spec = pl.BlockSpec((NC, BR, D), lambda i: (0, i, 0))
_cnt_spec = pl.BlockSpec((NC, BR, CNT_W), lambda i: (0, i, 0))
_w_spec = pl.BlockSpec((D, D), lambda i: (0, 0))
_b_spec = pl.BlockSpec((1, D), lambda i: (0, 0))

_tc_finish = pl.pallas_call(
    _tc_body,
    grid=(NBLK,),
    in_specs=[_sum_spec, _cnt_spec, _sum_spec, _cnt_spec, _sum_spec, _cnt_spec,
              _w_spec, _b_spec, _w_spec, _b_spec, _w_spec, _b_spec],
    out_specs=[pl.BlockSpec((BR, D), lambda i: (i, 0)),
               pl.BlockSpec((BR, D), lambda i: (i, 0))],
    out_shape=[jax.ShapeDtypeStruct((N_USER, D), jnp.float32),
               jax.ShapeDtypeStruct((N_ITEM, D), jnp.float32)],
)


def kernel(feat_user, feat_item, W_follows, b_follows, W_clicks, b_clicks,
           W_bought, b_bought, edge_index_follows, edge_index_clicks,
           edge_index_bought):
    def split_edges(ei):
        src = ei[0].reshape(NW, E_PER_W)
        dst = ei[1].reshape(NW, E_PER_W)
        spad = jnp.arange(PAD_E, dtype=jnp.int32) * 97 % N_USER
        dpad = DUMMY + (jnp.arange(PAD_E, dtype=jnp.int32) % 16)
        src = jnp.concatenate(
            [src, jnp.broadcast_to(spad, (NW, PAD_E))], axis=1)
        dst = jnp.concatenate(
            [dst, jnp.broadcast_to(dpad, (NW, PAD_E))], axis=1)
        return (src.reshape(NW, NCHUNKP, CHUNK), dst.reshape(NW, NCHUNKP, CHUNK))

    sf_, df_ = split_edges(edge_index_follows)
    sc_, dc_ = split_edges(edge_index_clicks)
    sb_, db_ = split_edges(edge_index_bought)

    z_s = jnp.zeros((RPT, D), jnp.float32)
    z_c = jnp.zeros((RPT, CNT_W), jnp.float32)
    ones_h = jnp.zeros((CHUNK, CNT_W), jnp.float32).at[:, 0].set(1.0)
    sum_f, cnt_f, sum_c, cnt_c, sum_b, cnt_b = _sc_agg(
        feat_user, feat_item, z_s, z_c, ones_h, sf_, df_, sc_, dc_, sb_, db_)

    h_user, h_item = _tc_finish(
        sum_f, cnt_f, sum_b, cnt_b, sum_c, cnt_c,
        W_follows, b_follows.reshape(1, D),
        W_bought, b_bought.reshape(1, D),
        W_clicks, b_clicks.reshape(1, D))
    return (h_user, h_item)


# speedup vs baseline: 2.0081x; 1.0335x over previous
import jax
import jax.numpy as jnp
from jax import lax
from jax.experimental import pallas as pl
from jax.experimental.pallas import tpu as pltpu
from jax.experimental.pallas import tpu_sc as plsc

N_USER = 10000
N_ITEM = 10000
E = 320000
D = 128
CNT_W = 8

NC, NS = 2, 16
NW = NC * NS
E_PER_W = E // NW
CHUNK = 88
NCHUNKP = 114
PAD_E = NCHUNKP * CHUNK - E_PER_W
NHALF = NCHUNKP // 2
DUMMY = N_USER
ACC_ROWS = N_USER + 16
RPT = 624
TAIL = N_USER - NS * RPT
TAIL0 = NS * RPT


def _sc_body(feat_user, feat_item, z_s, z_c, ones_h, src_f, dst_f, src_c, dst_c,
             src_b, dst_b,
             out_sf, out_cf, out_sc, out_cc, out_sb, out_cb,
             acc_s, acc_c, sidx, didx, rows0, rows1, ones, g0, g1, s0, s1):
    c = lax.axis_index("c")
    s = lax.axis_index("s")
    wid = c * NS + s
    row0 = pl.multiple_of(s * RPT, 8)

    pltpu.sync_copy(ones_h, ones)

    def run_relation(feat, src_r, dst_r, out_s, out_c):
        d1 = pltpu.async_copy(z_s, acc_s.at[pl.ds(row0, RPT)], g0)
        d2 = pltpu.async_copy(z_c, acc_c.at[pl.ds(row0, RPT)], g1)
        d3 = pltpu.async_copy(src_r.at[wid], sidx, s0)
        d4 = pltpu.async_copy(dst_r.at[wid], didx, s1)

        @pl.when(s == 0)
        def _zero_tail():
            pltpu.async_copy(
                z_s.at[pl.ds(0, TAIL)], acc_s.at[pl.ds(TAIL0, TAIL)], g0).wait()
            pltpu.async_copy(
                z_c.at[pl.ds(0, TAIL)], acc_c.at[pl.ds(TAIL0, TAIL)], g1).wait()

        d1.wait()
        d2.wait()
        d3.wait()
        d4.wait()
        plsc.subcore_barrier()

        def g_issue(buf, j, sem):
            pltpu.async_copy(feat.at[sidx.at[j]], buf, sem)

        def g_wait(buf, sem):
            pltpu.make_async_copy(feat.at[sidx.at[0]], buf, sem).wait()

        def s_issue(buf, j, sem):
            pltpu.async_copy(buf, acc_s.at[didx.at[j]], sem, add=True)
            pltpu.async_copy(ones, acc_c.at[didx.at[j]], sem, add=True)

        def s_wait(buf, sem):
            pltpu.make_async_copy(buf, acc_s.at[didx.at[0]], sem).wait()
            pltpu.make_async_copy(ones, acc_c.at[didx.at[0]], sem).wait()

        g_issue(rows0, 0, g0)
        g_issue(rows1, 1, g1)

        def step(i, carry):
            a = 2 * i
            g_wait(rows0, g0)
            s_issue(rows0, a, s0)
            g_wait(rows1, g1)
            s_issue(rows1, a + 1, s1)
            s_wait(rows0, s0)
            g_issue(rows0, a + 2, g0)
            s_wait(rows1, s1)
            g_issue(rows1, a + 3, g1)
            return carry

        lax.fori_loop(0, NHALF - 1, step, 0)
        g_wait(rows0, g0)
        s_issue(rows0, NCHUNKP - 2, s0)
        g_wait(rows1, g1)
        s_issue(rows1, NCHUNKP - 1, s1)
        s_wait(rows0, s0)
        s_wait(rows1, s1)

        plsc.subcore_barrier()
        o1 = pltpu.async_copy(acc_s.at[pl.ds(row0, RPT)],
                              out_s.at[c, pl.ds(row0, RPT)], g0)
        o2 = pltpu.async_copy(acc_c.at[pl.ds(row0, RPT)],
                              out_c.at[c, pl.ds(row0, RPT)], g1)

        @pl.when(s == 0)
        def _out_tail():
            pltpu.async_copy(acc_s.at[pl.ds(TAIL0, TAIL)],
                             out_s.at[c, pl.ds(TAIL0, TAIL)], s0).wait()
            pltpu.async_copy(acc_c.at[pl.ds(TAIL0, TAIL)],
                             out_c.at[c, pl.ds(TAIL0, TAIL)], s1).wait()

        o1.wait()
        o2.wait()
        plsc.subcore_barrier()

    run_relation(feat_user, src_f, dst_f, out_sf, out_cf)
    run_relation(feat_user, src_c, dst_c, out_sc, out_cc)
    run_relation(feat_item, src_b, dst_b, out_sb, out_cb)


_sc_agg = pl.kernel(
    _sc_body,
    out_type=[
        jax.ShapeDtypeStruct((NC, N_USER, D), jnp.float32),
        jax.ShapeDtypeStruct((NC, N_USER, CNT_W), jnp.float32),
        jax.ShapeDtypeStruct((NC, N_ITEM, D), jnp.float32),
        jax.ShapeDtypeStruct((NC, N_ITEM, CNT_W), jnp.float32),
        jax.ShapeDtypeStruct((NC, N_USER, D), jnp.float32),
        jax.ShapeDtypeStruct((NC, N_USER, CNT_W), jnp.float32),
    ],
    mesh=plsc.VectorSubcoreMesh(core_axis_name="c", subcore_axis_name="s"),
    compiler_params=pltpu.CompilerParams(use_tc_tiling_on_sc=False),
    scratch_types=[
        pltpu.VMEM_SHARED((ACC_ROWS, D), jnp.float32),
        pltpu.VMEM_SHARED((ACC_ROWS, CNT_W), jnp.float32),
        pltpu.VMEM((NCHUNKP, CHUNK), jnp.int32),
        pltpu.VMEM((NCHUNKP, CHUNK), jnp.int32),
        pltpu.VMEM((CHUNK, D), jnp.float32),
        pltpu.VMEM((CHUNK, D), jnp.float32),
        pltpu.VMEM((CHUNK, CNT_W), jnp.float32),
        pltpu.SemaphoreType.DMA,
        pltpu.SemaphoreType.DMA,
        pltpu.SemaphoreType.DMA,
        pltpu.SemaphoreType.DMA,
    ],
)

BR = 400
NBLK = N_USER // BR


def _tc_body(sf, cf, sb, cb, sc_r, cc, wf, bf, wb, bb, wc, bc, hu, hi):
    def rel(sref, cref, wref, bref):
        tot = sref[0] + sref[1]
        cnt = cref[0, :, 0:1] + cref[1, :, 0:1]
        mean = tot / jnp.maximum(cnt, 1.0)
        h = lax.dot_general(mean, wref[...], (((1,), (1,)), ((), ())),
                            preferred_element_type=jnp.float32,
                            precision=lax.Precision.HIGHEST) + bref[...]
        return jnp.where(cnt > 0, h, 0.0)

    hu[...] = rel(sf, cf, wf, bf) + rel(sb, cb, wb, bb)
    hi[...] = rel(sc_r, cc, wc, bc)


_sum_spec = pl.BlockSpec((NC, BR, D), lambda i: (0, i, 0))
_cnt_spec = pl.BlockSpec((NC, BR, CNT_W), lambda i: (0, i, 0))
_w_spec = pl.BlockSpec((D, D), lambda i: (0, 0))
_b_spec = pl.BlockSpec((1, D), lambda i: (0, 0))

_tc_finish = pl.pallas_call(
    _tc_body,
    grid=(NBLK,),
    in_specs=[_sum_spec, _cnt_spec, _sum_spec, _cnt_spec, _sum_spec, _cnt_spec,
              _w_spec, _b_spec, _w_spec, _b_spec, _w_spec, _b_spec],
    out_specs=[pl.BlockSpec((BR, D), lambda i: (i, 0)),
               pl.BlockSpec((BR, D), lambda i: (i, 0))],
    out_shape=[jax.ShapeDtypeStruct((N_USER, D), jnp.float32),
               jax.ShapeDtypeStruct((N_ITEM, D), jnp.float32)],
)


def kernel(feat_user, feat_item, W_follows, b_follows, W_clicks, b_clicks,
           W_bought, b_bought, edge_index_follows, edge_index_clicks,
           edge_index_bought):
    def split_edges(ei):
        src = ei[0].reshape(NW, E_PER_W)
        dst = ei[1].reshape(NW, E_PER_W)
        spad = jnp.arange(PAD_E, dtype=jnp.int32) * 97 % N_USER
        dpad = DUMMY + (jnp.arange(PAD_E, dtype=jnp.int32) % 16)
        src = jnp.concatenate(
            [src, jnp.broadcast_to(spad, (NW, PAD_E))], axis=1)
        dst = jnp.concatenate(
            [dst, jnp.broadcast_to(dpad, (NW, PAD_E))], axis=1)
        return (src.reshape(NW, NCHUNKP, CHUNK), dst.reshape(NW, NCHUNKP, CHUNK))

    sf_, df_ = split_edges(edge_index_follows)
    sc_, dc_ = split_edges(edge_index_clicks)
    sb_, db_ = split_edges(edge_index_bought)

    z_s = jnp.zeros((RPT, D), jnp.float32)
    z_c = jnp.zeros((RPT, CNT_W), jnp.float32)
    ones_h = jnp.zeros((CHUNK, CNT_W), jnp.float32).at[:, 0].set(1.0)
    sum_f, cnt_f, sum_c, cnt_c, sum_b, cnt_b = _sc_agg(
        feat_user, feat_item, z_s, z_c, ones_h, sf_, df_, sc_, dc_, sb_, db_)

    h_user, h_item = _tc_finish(
        sum_f, cnt_f, sum_b, cnt_b, sum_c, cnt_c,
        W_follows, b_follows.reshape(1, D),
        W_bought, b_bought.reshape(1, D),
        W_clicks, b_clicks.reshape(1, D))
    return (h_user, h_item)
